# baseline (speedup 1.0000x reference)
import functools

import jax
import jax.numpy as jnp
from jax import lax
from jax.experimental import pallas as pl
from jax.experimental.pallas import tpu as pltpu
from jax.experimental.pallas import tpu_sc as plsc

N_ROWS = 50000
D = 256
NUM_SEG = 128


_ROWS_PER_BLK = 1024
_N_BLKS = (N_ROWS + _ROWS_PER_BLK - 1) // _ROWS_PER_BLK


def _rowdot_body(x_ref, w_ref, o_ref):
    o_ref[...] = jnp.sum(x_ref[...] * w_ref[...], axis=1)


def _rowdot(x, w):
    return pl.pallas_call(
        _rowdot_body,
        grid=(_N_BLKS,),
        in_specs=[
            pl.BlockSpec((_ROWS_PER_BLK, D), lambda i: (i, 0)),
            pl.BlockSpec((1, D), lambda i: (0, 0)),
        ],
        out_specs=pl.BlockSpec((_ROWS_PER_BLK,), lambda i: (i,)),
        out_shape=jax.ShapeDtypeStruct((N_ROWS,), jnp.float32),
    )(x, w)


_NSUB = 16
_CHUNK = 3136
_VSTEPS = _CHUNK // 16
_LAST_BASE = N_ROWS - _CHUNK
_ACC_W = NUM_SEG + 16


@functools.partial(
    pl.kernel,
    mesh=plsc.VectorSubcoreMesh(
        core_axis_name="c", subcore_axis_name="s", num_cores=1),
    out_type=jax.ShapeDtypeStruct((NUM_SEG,), jnp.float32),
    compiler_params=pltpu.CompilerParams(needs_layout_passes=False),
    scratch_types=[
        pltpu.VMEM((_CHUNK,), jnp.float32),
        pltpu.VMEM((_CHUNK,), jnp.int32),
        pltpu.VMEM((16 * _ACC_W,), jnp.float32),
        pltpu.VMEM((16 * _ACC_W,), jnp.float32),
        pltpu.VMEM((NUM_SEG,), jnp.float32),
        pltpu.VMEM((NUM_SEG,), jnp.float32),
        pltpu.VMEM_SHARED((_NSUB * NUM_SEG,), jnp.float32),
        pltpu.VMEM_SHARED((_NSUB * NUM_SEG,), jnp.float32),
        pltpu.VMEM((_NSUB * NUM_SEG,), jnp.float32),
        pltpu.VMEM((_NSUB * NUM_SEG,), jnp.float32),
        pltpu.VMEM((16,), jnp.float32),
        pltpu.VMEM((NUM_SEG,), jnp.float32),
    ],
)
def _seg_mean(y_hbm, seg_hbm, b_hbm, out_hbm,
              y_v, s_v, acc, cnt, row_s, row_c,
              sh_s, sh_c, all_s, all_c, b_v, o_v):
    sid = lax.axis_index("s")
    own_lo = sid * _CHUNK
    base = jnp.minimum(own_lo, _LAST_BASE)
    pltpu.sync_copy(y_hbm.at[pl.ds(base, _CHUNK)], y_v)
    pltpu.sync_copy(seg_hbm.at[pl.ds(base, _CHUNK)], s_v)

    zeros = jnp.zeros((16,), jnp.float32)
    ones = jnp.ones((16,), jnp.float32)
    lane = lax.iota(jnp.int32, 16)
    lane_off = lane * _ACC_W

    for j in range(16 * _ACC_W // 16):
        acc[pl.ds(j * 16, 16)] = zeros
        cnt[pl.ds(j * 16, 16)] = zeros

    def body(k, carry):
        yv = y_v[pl.ds(k * 16, 16)]
        sv = s_v[pl.ds(k * 16, 16)]
        g = base + k * 16 + lane
        m = (g >= own_lo) & (g < N_ROWS)
        idx = lane_off + sv
        plsc.addupdate_scatter(acc, [idx], yv, mask=m)
        plsc.addupdate_scatter(cnt, [idx], ones, mask=m)
        return carry

    lax.fori_loop(0, _VSTEPS, body, 0)

    for j in range(NUM_SEG // 16):
        s = acc[pl.ds(j * 16, 16)]
        c = cnt[pl.ds(j * 16, 16)]
        for i in range(1, 16):
            s = s + acc[pl.ds(i * _ACC_W + j * 16, 16)]
            c = c + cnt[pl.ds(i * _ACC_W + j * 16, 16)]
        row_s[pl.ds(j * 16, 16)] = s
        row_c[pl.ds(j * 16, 16)] = c

    pltpu.sync_copy(row_s, sh_s.at[pl.ds(sid * NUM_SEG, NUM_SEG)])
    pltpu.sync_copy(row_c, sh_c.at[pl.ds(sid * NUM_SEG, NUM_SEG)])
    plsc.subcore_barrier()

    @pl.when(sid == 0)
    def _():
        pltpu.sync_copy(sh_s, all_s)
        pltpu.sync_copy(sh_c, all_c)
        pltpu.sync_copy(b_hbm, b_v)
        bvec = b_v[...]
        for j in range(NUM_SEG // 16):
            s = all_s[pl.ds(j * 16, 16)]
            c = all_c[pl.ds(j * 16, 16)]
            for i in range(1, _NSUB):
                s = s + all_s[pl.ds(i * NUM_SEG + j * 16, 16)]
                c = c + all_c[pl.ds(i * NUM_SEG + j * 16, 16)]
            o_v[pl.ds(j * 16, 16)] = s / jnp.maximum(c, 1.0) + bvec
        pltpu.sync_copy(o_v, out_hbm)


def kernel(x, segment_ids, W, b):
    y = _rowdot(x, W.astype(jnp.float32))
    seg = segment_ids.astype(jnp.int32)
    b16 = jnp.broadcast_to(b.astype(jnp.float32), (16,))
    return _seg_mean(y, seg, b16)


# speedup vs baseline: 1.3739x; 1.3739x over previous
import functools

import jax
import jax.numpy as jnp
from jax import lax
from jax.experimental import pallas as pl
from jax.experimental.pallas import tpu as pltpu
from jax.experimental.pallas import tpu_sc as plsc

N_ROWS = 50000
D = 256
NUM_SEG = 128


_ROWS_PER_BLK = 2048
_N_BLKS = (N_ROWS + _ROWS_PER_BLK - 1) // _ROWS_PER_BLK
_N_PAD = _N_BLKS * _ROWS_PER_BLK


def _rowdot_body(x_ref, w_ref, o_ref):
    s = jnp.sum(x_ref[...] * w_ref[...], axis=1)
    o_ref[...] = s.reshape(_ROWS_PER_BLK // 128, 128)


def _rowdot(x, w):
    return pl.pallas_call(
        _rowdot_body,
        grid=(_N_BLKS,),
        in_specs=[
            pl.BlockSpec((_ROWS_PER_BLK, D), lambda i: (i, 0)),
            pl.BlockSpec((1, D), lambda i: (0, 0)),
        ],
        out_specs=pl.BlockSpec((_ROWS_PER_BLK // 128, 128), lambda i: (i, 0)),
        out_shape=jax.ShapeDtypeStruct((_N_PAD // 128, 128), jnp.float32),
    )(x, w)


_NSUB = 16
_CHUNK = _N_PAD // _NSUB
_VSTEPS = _CHUNK // 16
_ACC_W = NUM_SEG + 16


@functools.partial(
    pl.kernel,
    mesh=plsc.VectorSubcoreMesh(
        core_axis_name="c", subcore_axis_name="s", num_cores=1),
    out_type=jax.ShapeDtypeStruct((NUM_SEG,), jnp.float32),
    compiler_params=pltpu.CompilerParams(needs_layout_passes=False),
    scratch_types=[
        pltpu.VMEM((_CHUNK,), jnp.float32),
        pltpu.VMEM((_CHUNK,), jnp.int32),
        pltpu.VMEM((16 * _ACC_W,), jnp.float32),
        pltpu.VMEM((16 * _ACC_W,), jnp.float32),
        pltpu.VMEM((NUM_SEG,), jnp.float32),
        pltpu.VMEM((NUM_SEG,), jnp.float32),
        pltpu.VMEM_SHARED((_NSUB * NUM_SEG,), jnp.float32),
        pltpu.VMEM_SHARED((_NSUB * NUM_SEG,), jnp.float32),
        pltpu.VMEM((_NSUB * NUM_SEG,), jnp.float32),
        pltpu.VMEM((_NSUB * NUM_SEG,), jnp.float32),
        pltpu.VMEM((16,), jnp.float32),
        pltpu.VMEM((NUM_SEG,), jnp.float32),
    ],
)
def _seg_mean(y_hbm, seg_hbm, b_hbm, out_hbm,
              y_v, s_v, acc, cnt, row_s, row_c,
              sh_s, sh_c, all_s, all_c, b_v, o_v):
    sid = lax.axis_index("s")
    base = sid * _CHUNK
    pltpu.sync_copy(y_hbm.at[pl.ds(base, _CHUNK)], y_v)
    pltpu.sync_copy(seg_hbm.at[pl.ds(base, _CHUNK)], s_v)

    zeros = jnp.zeros((16,), jnp.float32)
    ones = jnp.ones((16,), jnp.float32)
    lane = lax.iota(jnp.int32, 16)
    lane_off = lane * _ACC_W

    for j in range(16 * _ACC_W // 16):
        acc[pl.ds(j * 16, 16)] = zeros
        cnt[pl.ds(j * 16, 16)] = zeros

    def body(k, carry):
        yv = y_v[pl.ds(k * 16, 16)]
        sv = s_v[pl.ds(k * 16, 16)]
        g = base + k * 16 + lane
        m = g < N_ROWS
        idx = lane_off + sv
        plsc.addupdate_scatter(acc, [idx], yv, mask=m)
        plsc.addupdate_scatter(cnt, [idx], ones, mask=m)
        return carry

    lax.fori_loop(0, _VSTEPS, body, 0)

    for j in range(NUM_SEG // 16):
        s = acc[pl.ds(j * 16, 16)]
        c = cnt[pl.ds(j * 16, 16)]
        for i in range(1, 16):
            s = s + acc[pl.ds(i * _ACC_W + j * 16, 16)]
            c = c + cnt[pl.ds(i * _ACC_W + j * 16, 16)]
        row_s[pl.ds(j * 16, 16)] = s
        row_c[pl.ds(j * 16, 16)] = c

    pltpu.sync_copy(row_s, sh_s.at[pl.ds(sid * NUM_SEG, NUM_SEG)])
    pltpu.sync_copy(row_c, sh_c.at[pl.ds(sid * NUM_SEG, NUM_SEG)])
    plsc.subcore_barrier()

    @pl.when(sid == 0)
    def _():
        pltpu.sync_copy(sh_s, all_s)
        pltpu.sync_copy(sh_c, all_c)
        pltpu.sync_copy(b_hbm, b_v)
        bvec = b_v[...]
        for j in range(NUM_SEG // 16):
            s = all_s[pl.ds(j * 16, 16)]
            c = all_c[pl.ds(j * 16, 16)]
            for i in range(1, _NSUB):
                s = s + all_s[pl.ds(i * NUM_SEG + j * 16, 16)]
                c = c + all_c[pl.ds(i * NUM_SEG + j * 16, 16)]
            o_v[pl.ds(j * 16, 16)] = s / jnp.maximum(c, 1.0) + bvec
        pltpu.sync_copy(o_v, out_hbm)


def kernel(x, segment_ids, W, b):
    y = _rowdot(x, W.astype(jnp.float32)).reshape(_N_PAD)
    seg = jnp.pad(segment_ids.astype(jnp.int32), (0, _N_PAD - N_ROWS))
    b16 = jnp.broadcast_to(b.astype(jnp.float32), (16,))
    return _seg_mean(y, seg, b16)


# speedup vs baseline: 1.5660x; 1.1398x over previous
import functools

import jax
import jax.numpy as jnp
from jax import lax
from jax.experimental import pallas as pl
from jax.experimental.pallas import tpu as pltpu
from jax.experimental.pallas import tpu_sc as plsc

N_ROWS = 50000
D = 256
NUM_SEG = 128


_ROWS_PER_BLK = 4096
_N_BLKS = (N_ROWS + _ROWS_PER_BLK - 1) // _ROWS_PER_BLK
_N_PAD = _N_BLKS * _ROWS_PER_BLK


def _rowdot_body(x_ref, w_ref, o_ref):
    s = jnp.sum(x_ref[...] * w_ref[...], axis=1)
    o_ref[...] = s.reshape(_ROWS_PER_BLK // 128, 128)


def _rowdot(x, w):
    return pl.pallas_call(
        _rowdot_body,
        grid=(_N_BLKS,),
        in_specs=[
            pl.BlockSpec((_ROWS_PER_BLK, D), lambda i: (i, 0)),
            pl.BlockSpec((1, D), lambda i: (0, 0)),
        ],
        out_specs=pl.BlockSpec((_ROWS_PER_BLK // 128, 128), lambda i: (i, 0)),
        out_shape=jax.ShapeDtypeStruct((_N_PAD // 128, 128), jnp.float32),
    )(x, w)


_NSUB = 16
_CHUNK = _N_PAD // _NSUB
_VSTEPS = _CHUNK // 16
_ACC_W = NUM_SEG + 16


@functools.partial(
    pl.kernel,
    mesh=plsc.VectorSubcoreMesh(
        core_axis_name="c", subcore_axis_name="s", num_cores=1),
    out_type=jax.ShapeDtypeStruct((NUM_SEG,), jnp.float32),
    compiler_params=pltpu.CompilerParams(needs_layout_passes=False),
    scratch_types=[
        pltpu.VMEM((_CHUNK,), jnp.float32),
        pltpu.VMEM((_CHUNK,), jnp.int32),
        pltpu.VMEM((16 * _ACC_W,), jnp.float32),
        pltpu.VMEM((16 * _ACC_W,), jnp.float32),
        pltpu.VMEM((NUM_SEG,), jnp.float32),
        pltpu.VMEM((NUM_SEG,), jnp.float32),
        pltpu.VMEM_SHARED((_NSUB * NUM_SEG,), jnp.float32),
        pltpu.VMEM_SHARED((_NSUB * NUM_SEG,), jnp.float32),
        pltpu.VMEM((_NSUB * NUM_SEG,), jnp.float32),
        pltpu.VMEM((_NSUB * NUM_SEG,), jnp.float32),
        pltpu.VMEM((16,), jnp.float32),
        pltpu.VMEM((NUM_SEG,), jnp.float32),
    ],
)
def _seg_mean(y_hbm, seg_hbm, b_hbm, out_hbm,
              y_v, s_v, acc, cnt, row_s, row_c,
              sh_s, sh_c, all_s, all_c, b_v, o_v):
    sid = lax.axis_index("s")
    base = sid * _CHUNK
    pltpu.sync_copy(y_hbm.at[pl.ds(base, _CHUNK)], y_v)
    pltpu.sync_copy(seg_hbm.at[pl.ds(base, _CHUNK)], s_v)

    zeros = jnp.zeros((16,), jnp.float32)
    ones = jnp.ones((16,), jnp.float32)
    lane = lax.iota(jnp.int32, 16)
    lane_off = lane * _ACC_W

    for j in range(16 * _ACC_W // 16):
        acc[pl.ds(j * 16, 16)] = zeros
        cnt[pl.ds(j * 16, 16)] = zeros

    def body(k, carry):
        yv = y_v[pl.ds(k * 16, 16)]
        sv = s_v[pl.ds(k * 16, 16)]
        g = base + k * 16 + lane
        m = g < N_ROWS
        idx = lane_off + sv
        plsc.addupdate_scatter(acc, [idx], yv, mask=m)
        plsc.addupdate_scatter(cnt, [idx], ones, mask=m)
        return carry

    lax.fori_loop(0, _VSTEPS, body, 0)

    for j in range(NUM_SEG // 16):
        s = acc[pl.ds(j * 16, 16)]
        c = cnt[pl.ds(j * 16, 16)]
        for i in range(1, 16):
            s = s + acc[pl.ds(i * _ACC_W + j * 16, 16)]
            c = c + cnt[pl.ds(i * _ACC_W + j * 16, 16)]
        row_s[pl.ds(j * 16, 16)] = s
        row_c[pl.ds(j * 16, 16)] = c

    pltpu.sync_copy(row_s, sh_s.at[pl.ds(sid * NUM_SEG, NUM_SEG)])
    pltpu.sync_copy(row_c, sh_c.at[pl.ds(sid * NUM_SEG, NUM_SEG)])
    plsc.subcore_barrier()

    @pl.when(sid == 0)
    def _():
        pltpu.sync_copy(sh_s, all_s)
        pltpu.sync_copy(sh_c, all_c)
        pltpu.sync_copy(b_hbm, b_v)
        bvec = b_v[...]
        for j in range(NUM_SEG // 16):
            s = all_s[pl.ds(j * 16, 16)]
            c = all_c[pl.ds(j * 16, 16)]
            for i in range(1, _NSUB):
                s = s + all_s[pl.ds(i * NUM_SEG + j * 16, 16)]
                c = c + all_c[pl.ds(i * NUM_SEG + j * 16, 16)]
            o_v[pl.ds(j * 16, 16)] = s / jnp.maximum(c, 1.0) + bvec
        pltpu.sync_copy(o_v, out_hbm)


def kernel(x, segment_ids, W, b):
    y = _rowdot(x, W.astype(jnp.float32)).reshape(_N_PAD)
    seg = jnp.pad(segment_ids.astype(jnp.int32), (0, _N_PAD - N_ROWS))
    b16 = jnp.broadcast_to(b.astype(jnp.float32), (16,))
    return _seg_mean(y, seg, b16)


# speedup vs baseline: 1.6118x; 1.0292x over previous
import functools

import jax
import jax.numpy as jnp
from jax import lax
from jax.experimental import pallas as pl
from jax.experimental.pallas import tpu as pltpu
from jax.experimental.pallas import tpu_sc as plsc

N_ROWS = 50000
D = 256
NUM_SEG = 128


_ROWS_PER_BLK = 8192
_N_BLKS = (N_ROWS + _ROWS_PER_BLK - 1) // _ROWS_PER_BLK
_N_PAD = _N_BLKS * _ROWS_PER_BLK


def _rowdot_body(x_ref, w_ref, o_ref):
    s = jnp.sum(x_ref[...] * w_ref[...], axis=1)
    o_ref[...] = s.reshape(_ROWS_PER_BLK // 128, 128)


def _rowdot(x, w):
    return pl.pallas_call(
        _rowdot_body,
        grid=(_N_BLKS,),
        in_specs=[
            pl.BlockSpec((_ROWS_PER_BLK, D), lambda i: (i, 0)),
            pl.BlockSpec((1, D), lambda i: (0, 0)),
        ],
        out_specs=pl.BlockSpec((_ROWS_PER_BLK // 128, 128), lambda i: (i, 0)),
        out_shape=jax.ShapeDtypeStruct((_N_PAD // 128, 128), jnp.float32),
    )(x, w)


_NSUB = 16
_CHUNK = _N_PAD // _NSUB
_VSTEPS = _CHUNK // 16
_ACC_W = NUM_SEG + 16


@functools.partial(
    pl.kernel,
    mesh=plsc.VectorSubcoreMesh(
        core_axis_name="c", subcore_axis_name="s", num_cores=1),
    out_type=jax.ShapeDtypeStruct((NUM_SEG,), jnp.float32),
    compiler_params=pltpu.CompilerParams(needs_layout_passes=False),
    scratch_types=[
        pltpu.VMEM((_CHUNK,), jnp.float32),
        pltpu.VMEM((_CHUNK,), jnp.int32),
        pltpu.VMEM((16 * _ACC_W,), jnp.float32),
        pltpu.VMEM((16 * _ACC_W,), jnp.float32),
        pltpu.VMEM((NUM_SEG,), jnp.float32),
        pltpu.VMEM((NUM_SEG,), jnp.float32),
        pltpu.VMEM_SHARED((_NSUB * NUM_SEG,), jnp.float32),
        pltpu.VMEM_SHARED((_NSUB * NUM_SEG,), jnp.float32),
        pltpu.VMEM((_NSUB * NUM_SEG,), jnp.float32),
        pltpu.VMEM((_NSUB * NUM_SEG,), jnp.float32),
        pltpu.VMEM((16,), jnp.float32),
        pltpu.VMEM((NUM_SEG,), jnp.float32),
    ],
)
def _seg_mean(y_hbm, seg_hbm, b_hbm, out_hbm,
              y_v, s_v, acc, cnt, row_s, row_c,
              sh_s, sh_c, all_s, all_c, b_v, o_v):
    sid = lax.axis_index("s")
    base = sid * _CHUNK
    pltpu.sync_copy(y_hbm.at[pl.ds(base, _CHUNK)], y_v)
    pltpu.sync_copy(seg_hbm.at[pl.ds(base, _CHUNK)], s_v)

    zeros = jnp.zeros((16,), jnp.float32)
    ones = jnp.ones((16,), jnp.float32)
    lane = lax.iota(jnp.int32, 16)
    lane_off = lane * _ACC_W

    for j in range(16 * _ACC_W // 16):
        acc[pl.ds(j * 16, 16)] = zeros
        cnt[pl.ds(j * 16, 16)] = zeros

    def body(k, carry):
        yv = y_v[pl.ds(k * 16, 16)]
        sv = s_v[pl.ds(k * 16, 16)]
        g = base + k * 16 + lane
        m = g < N_ROWS
        idx = lane_off + sv
        plsc.addupdate_scatter(acc, [idx], yv, mask=m)
        plsc.addupdate_scatter(cnt, [idx], ones, mask=m)
        return carry

    lax.fori_loop(0, _VSTEPS, body, 0)

    for j in range(NUM_SEG // 16):
        s = acc[pl.ds(j * 16, 16)]
        c = cnt[pl.ds(j * 16, 16)]
        for i in range(1, 16):
            s = s + acc[pl.ds(i * _ACC_W + j * 16, 16)]
            c = c + cnt[pl.ds(i * _ACC_W + j * 16, 16)]
        row_s[pl.ds(j * 16, 16)] = s
        row_c[pl.ds(j * 16, 16)] = c

    pltpu.sync_copy(row_s, sh_s.at[pl.ds(sid * NUM_SEG, NUM_SEG)])
    pltpu.sync_copy(row_c, sh_c.at[pl.ds(sid * NUM_SEG, NUM_SEG)])
    plsc.subcore_barrier()

    @pl.when(sid == 0)
    def _():
        pltpu.sync_copy(sh_s, all_s)
        pltpu.sync_copy(sh_c, all_c)
        pltpu.sync_copy(b_hbm, b_v)
        bvec = b_v[...]
        for j in range(NUM_SEG // 16):
            s = all_s[pl.ds(j * 16, 16)]
            c = all_c[pl.ds(j * 16, 16)]
            for i in range(1, _NSUB):
                s = s + all_s[pl.ds(i * NUM_SEG + j * 16, 16)]
                c = c + all_c[pl.ds(i * NUM_SEG + j * 16, 16)]
            o_v[pl.ds(j * 16, 16)] = s / jnp.maximum(c, 1.0) + bvec
        pltpu.sync_copy(o_v, out_hbm)


def kernel(x, segment_ids, W, b):
    y = _rowdot(x, W.astype(jnp.float32)).reshape(_N_PAD)
    seg = jnp.pad(segment_ids.astype(jnp.int32), (0, _N_PAD - N_ROWS))
    b16 = jnp.broadcast_to(b.astype(jnp.float32), (16,))
    return _seg_mean(y, seg, b16)


# speedup vs baseline: 1.6822x; 1.0437x over previous
import functools

import jax
import jax.numpy as jnp
from jax import lax
from jax.experimental import pallas as pl
from jax.experimental.pallas import tpu as pltpu
from jax.experimental.pallas import tpu_sc as plsc

N_ROWS = 50000
D = 256
NUM_SEG = 128


_ROWS_PER_BLK = 10240
_N_BLKS = (N_ROWS + _ROWS_PER_BLK - 1) // _ROWS_PER_BLK
_N_PAD = _N_BLKS * _ROWS_PER_BLK


def _rowdot_body(x_ref, w_ref, o_ref):
    s = jnp.sum(x_ref[...] * w_ref[...], axis=1)
    o_ref[...] = s.reshape(_ROWS_PER_BLK // 128, 128)


def _rowdot(x, w):
    return pl.pallas_call(
        _rowdot_body,
        grid=(_N_BLKS,),
        in_specs=[
            pl.BlockSpec((_ROWS_PER_BLK, D), lambda i: (i, 0)),
            pl.BlockSpec((1, D), lambda i: (0, 0)),
        ],
        out_specs=pl.BlockSpec((_ROWS_PER_BLK // 128, 128), lambda i: (i, 0)),
        out_shape=jax.ShapeDtypeStruct((_N_PAD // 128, 128), jnp.float32),
    )(x, w)


_NSUB = 16
_CHUNK = _N_PAD // _NSUB
_VSTEPS = _CHUNK // 16
_ACC_W = NUM_SEG + 16


@functools.partial(
    pl.kernel,
    mesh=plsc.VectorSubcoreMesh(
        core_axis_name="c", subcore_axis_name="s", num_cores=1),
    out_type=jax.ShapeDtypeStruct((NUM_SEG,), jnp.float32),
    compiler_params=pltpu.CompilerParams(needs_layout_passes=False),
    scratch_types=[
        pltpu.VMEM((_CHUNK,), jnp.float32),
        pltpu.VMEM((_CHUNK,), jnp.int32),
        pltpu.VMEM((16 * _ACC_W,), jnp.float32),
        pltpu.VMEM((16 * _ACC_W,), jnp.float32),
        pltpu.VMEM((NUM_SEG,), jnp.float32),
        pltpu.VMEM((NUM_SEG,), jnp.float32),
        pltpu.VMEM_SHARED((_NSUB * NUM_SEG,), jnp.float32),
        pltpu.VMEM_SHARED((_NSUB * NUM_SEG,), jnp.float32),
        pltpu.VMEM((_NSUB * NUM_SEG,), jnp.float32),
        pltpu.VMEM((_NSUB * NUM_SEG,), jnp.float32),
        pltpu.VMEM((16,), jnp.float32),
        pltpu.VMEM((NUM_SEG,), jnp.float32),
    ],
)
def _seg_mean(y_hbm, seg_hbm, b_hbm, out_hbm,
              y_v, s_v, acc, cnt, row_s, row_c,
              sh_s, sh_c, all_s, all_c, b_v, o_v):
    sid = lax.axis_index("s")
    base = sid * _CHUNK
    pltpu.sync_copy(y_hbm.at[pl.ds(base, _CHUNK)], y_v)
    pltpu.sync_copy(seg_hbm.at[pl.ds(base, _CHUNK)], s_v)

    zeros = jnp.zeros((16,), jnp.float32)
    ones = jnp.ones((16,), jnp.float32)
    lane = lax.iota(jnp.int32, 16)
    lane_off = lane * _ACC_W

    for j in range(16 * _ACC_W // 16):
        acc[pl.ds(j * 16, 16)] = zeros
        cnt[pl.ds(j * 16, 16)] = zeros

    def body(k, carry):
        yv = y_v[pl.ds(k * 16, 16)]
        sv = s_v[pl.ds(k * 16, 16)]
        idx = lane_off + sv
        plsc.addupdate_scatter(acc, [idx], yv)
        plsc.addupdate_scatter(cnt, [idx], ones)
        return carry

    lax.fori_loop(0, _VSTEPS, body, 0)

    for j in range(NUM_SEG // 16):
        s = acc[pl.ds(j * 16, 16)]
        c = cnt[pl.ds(j * 16, 16)]
        for i in range(1, 16):
            s = s + acc[pl.ds(i * _ACC_W + j * 16, 16)]
            c = c + cnt[pl.ds(i * _ACC_W + j * 16, 16)]
        row_s[pl.ds(j * 16, 16)] = s
        row_c[pl.ds(j * 16, 16)] = c

    pltpu.sync_copy(row_s, sh_s.at[pl.ds(sid * NUM_SEG, NUM_SEG)])
    pltpu.sync_copy(row_c, sh_c.at[pl.ds(sid * NUM_SEG, NUM_SEG)])
    plsc.subcore_barrier()

    @pl.when(sid == 0)
    def _():
        pltpu.sync_copy(sh_s, all_s)
        pltpu.sync_copy(sh_c, all_c)
        pltpu.sync_copy(b_hbm, b_v.at[pl.ds(0, 1)])
        bvec = plsc.load_gather(b_v, [jnp.zeros((16,), jnp.int32)])
        for j in range(NUM_SEG // 16):
            s = all_s[pl.ds(j * 16, 16)]
            c = all_c[pl.ds(j * 16, 16)]
            for i in range(1, _NSUB):
                s = s + all_s[pl.ds(i * NUM_SEG + j * 16, 16)]
                c = c + all_c[pl.ds(i * NUM_SEG + j * 16, 16)]
            o_v[pl.ds(j * 16, 16)] = s / jnp.maximum(c, 1.0) + bvec
        pltpu.sync_copy(o_v, out_hbm)


def kernel(x, segment_ids, W, b):
    y = _rowdot(x, W.astype(jnp.float32)).reshape(_N_PAD)
    seg = jnp.pad(segment_ids.astype(jnp.int32), (0, _N_PAD - N_ROWS),
                  constant_values=NUM_SEG)
    return _seg_mean(y, seg, b.astype(jnp.float32))


# speedup vs baseline: 1.7062x; 1.0143x over previous
import functools

import jax
import jax.numpy as jnp
from jax import lax
from jax.experimental import pallas as pl
from jax.experimental.pallas import tpu as pltpu
from jax.experimental.pallas import tpu_sc as plsc

N_ROWS = 50000
D = 256
NUM_SEG = 128


_ROWS_PER_BLK = 10240
_N_BLKS = (N_ROWS + _ROWS_PER_BLK - 1) // _ROWS_PER_BLK
_N_PAD = _N_BLKS * _ROWS_PER_BLK


_BLK_SUB = _ROWS_PER_BLK // 128


def _rowdot_body(x_ref, w_ref, s_ref, o_ref, o2_ref):
    s = jnp.sum(x_ref[...] * w_ref[...], axis=1)
    o_ref[...] = s.reshape(_BLK_SUB, 128)
    pid = pl.program_id(0)
    g = (pid * _ROWS_PER_BLK
         + jax.lax.broadcasted_iota(jnp.int32, (_BLK_SUB, 128), 0) * 128
         + jax.lax.broadcasted_iota(jnp.int32, (_BLK_SUB, 128), 1))
    sblk = s_ref[...].reshape(_BLK_SUB, 128)
    o2_ref[...] = jnp.where(g < N_ROWS, sblk, NUM_SEG)


def _rowdot(x, w, seg):
    return pl.pallas_call(
        _rowdot_body,
        grid=(_N_BLKS,),
        in_specs=[
            pl.BlockSpec((_ROWS_PER_BLK, D), lambda i: (i, 0)),
            pl.BlockSpec((1, D), lambda i: (0, 0)),
            pl.BlockSpec((_ROWS_PER_BLK,), lambda i: (i,)),
        ],
        out_specs=[
            pl.BlockSpec((_BLK_SUB, 128), lambda i: (i, 0)),
            pl.BlockSpec((_BLK_SUB, 128), lambda i: (i, 0)),
        ],
        out_shape=[
            jax.ShapeDtypeStruct((_N_PAD // 128, 128), jnp.float32),
            jax.ShapeDtypeStruct((_N_PAD // 128, 128), jnp.int32),
        ],
    )(x, w, seg)


_NSUB = 16
_CHUNK = _N_PAD // _NSUB
_VSTEPS = _CHUNK // 16
_UNROLL = 4
_ACC_W = NUM_SEG + 16


@functools.partial(
    pl.kernel,
    mesh=plsc.VectorSubcoreMesh(
        core_axis_name="c", subcore_axis_name="s", num_cores=1),
    out_type=jax.ShapeDtypeStruct((NUM_SEG,), jnp.float32),
    compiler_params=pltpu.CompilerParams(needs_layout_passes=False),
    scratch_types=[
        pltpu.VMEM((_CHUNK,), jnp.float32),
        pltpu.VMEM((_CHUNK,), jnp.int32),
        pltpu.VMEM((16 * _ACC_W,), jnp.float32),
        pltpu.VMEM((16 * _ACC_W,), jnp.float32),
        pltpu.VMEM((NUM_SEG,), jnp.float32),
        pltpu.VMEM((NUM_SEG,), jnp.float32),
        pltpu.VMEM_SHARED((_NSUB * NUM_SEG,), jnp.float32),
        pltpu.VMEM_SHARED((_NSUB * NUM_SEG,), jnp.float32),
        pltpu.VMEM((_NSUB * NUM_SEG,), jnp.float32),
        pltpu.VMEM((_NSUB * NUM_SEG,), jnp.float32),
        pltpu.VMEM((16,), jnp.float32),
        pltpu.VMEM((NUM_SEG,), jnp.float32),
    ],
)
def _seg_mean(y_hbm, seg_hbm, b_hbm, out_hbm,
              y_v, s_v, acc, cnt, row_s, row_c,
              sh_s, sh_c, all_s, all_c, b_v, o_v):
    sid = lax.axis_index("s")
    base = sid * _CHUNK
    pltpu.sync_copy(y_hbm.at[pl.ds(base, _CHUNK)], y_v)
    pltpu.sync_copy(seg_hbm.at[pl.ds(base, _CHUNK)], s_v)

    zeros = jnp.zeros((16,), jnp.float32)
    ones = jnp.ones((16,), jnp.float32)
    lane = lax.iota(jnp.int32, 16)
    lane_off = lane * _ACC_W

    for j in range(16 * _ACC_W // 16):
        acc[pl.ds(j * 16, 16)] = zeros
        cnt[pl.ds(j * 16, 16)] = zeros

    def body(k, carry):
        for u in range(_UNROLL):
            yv = y_v[pl.ds(k * (16 * _UNROLL) + u * 16, 16)]
            sv = s_v[pl.ds(k * (16 * _UNROLL) + u * 16, 16)]
            idx = lane_off + sv
            plsc.addupdate_scatter(acc, [idx], yv)
            plsc.addupdate_scatter(cnt, [idx], ones)
        return carry

    lax.fori_loop(0, _VSTEPS // _UNROLL, body, 0)

    for j in range(NUM_SEG // 16):
        s = acc[pl.ds(j * 16, 16)]
        c = cnt[pl.ds(j * 16, 16)]
        for i in range(1, 16):
            s = s + acc[pl.ds(i * _ACC_W + j * 16, 16)]
            c = c + cnt[pl.ds(i * _ACC_W + j * 16, 16)]
        row_s[pl.ds(j * 16, 16)] = s
        row_c[pl.ds(j * 16, 16)] = c

    pltpu.sync_copy(row_s, sh_s.at[pl.ds(sid * NUM_SEG, NUM_SEG)])
    pltpu.sync_copy(row_c, sh_c.at[pl.ds(sid * NUM_SEG, NUM_SEG)])
    plsc.subcore_barrier()

    @pl.when(sid == 0)
    def _():
        pltpu.sync_copy(sh_s, all_s)
        pltpu.sync_copy(sh_c, all_c)
        pltpu.sync_copy(b_hbm, b_v.at[pl.ds(0, 1)])
        bvec = plsc.load_gather(b_v, [jnp.zeros((16,), jnp.int32)])
        for j in range(NUM_SEG // 16):
            s = all_s[pl.ds(j * 16, 16)]
            c = all_c[pl.ds(j * 16, 16)]
            for i in range(1, _NSUB):
                s = s + all_s[pl.ds(i * NUM_SEG + j * 16, 16)]
                c = c + all_c[pl.ds(i * NUM_SEG + j * 16, 16)]
            o_v[pl.ds(j * 16, 16)] = s / jnp.maximum(c, 1.0) + bvec
        pltpu.sync_copy(o_v, out_hbm)


def kernel(x, segment_ids, W, b):
    y2, seg2 = _rowdot(x, W.astype(jnp.float32), segment_ids.astype(jnp.int32))
    return _seg_mean(y2.reshape(_N_PAD), seg2.reshape(_N_PAD),
                     b.astype(jnp.float32))
